# baseline (speedup 1.0000x reference)
import functools
import math

import jax
import jax.numpy as jnp
from jax import lax
from jax.experimental import pallas as pl
from jax.experimental.pallas import tpu as pltpu
from jax.experimental.pallas import tpu_sc as plsc

B = 512
L = 50
NP = 56
EPG = 128
GS = 8
GN = GS * NP
H1, C1 = 7, 256
H2, C2 = 7, 128
D1 = H1 * C1
D2 = H2 * C2
NEG = -1e30


def _sc_gather(table, ids):
    m = ids.shape[1]
    window = 128
    nwin = m // window
    mesh = plsc.VectorSubcoreMesh(core_axis_name="core", subcore_axis_name="subcore")

    @pl.kernel(
        out_type=jax.ShapeDtypeStruct((m, 128), table.dtype),
        mesh=mesh,
    )
    def gather_kernel(x_hbm, i_hbm, o_hbm):
        def body(i_vmem, o_vmem):
            pltpu.sync_copy(x_hbm.at[i_vmem.at[0]], o_vmem)

        pltpu.emit_pipeline(
            body,
            grid=(2, nwin // 2),
            in_specs=[pl.BlockSpec((1, window),
                                   lambda c, i: (0, c * (nwin // 2) + i))],
            out_specs=[pl.BlockSpec((window, 128),
                                    lambda c, i: (c * (nwin // 2) + i, 0))],
            core_axis_name=("core", "subcore"),
            dimension_semantics=(pltpu.PARALLEL, pltpu.PARALLEL),
        )(i_hbm, o_hbm)

    return gather_kernel(table, ids)


def _attention(logits_t, self_logits, xls, xself, mask7, heads, chead, sel):
    lh7 = jnp.dot(sel, logits_t, preferred_element_type=jnp.float32)
    slt = jnp.concatenate([self_logits] * heads, axis=0)
    ls7 = jnp.sum(slt * sel, axis=1, keepdims=True)
    cand = jnp.where(mask7, lh7, NEG)
    m = jnp.maximum(jnp.max(cand, axis=1, keepdims=True), ls7)
    z = jnp.exp(cand - m)
    zs = jnp.exp(ls7 - m)
    inv = 1.0 / (jnp.sum(z, axis=1, keepdims=True) + zs + 1e-16)
    outs = []
    for h in range(heads):
        rs = slice(h * NP, (h + 1) * NP)
        sl = slice(h * chead, (h + 1) * chead)
        agg = jnp.dot(z[rs], xls[:, sl], preferred_element_type=jnp.float32)
        outs.append(agg * inv[rs] + (zs[rs] * inv[rs]) * xself[:, sl])
    return jnp.concatenate(outs, axis=1)


def _gnn_body(
    x_ref, t_ref, p_ref, s2_ref, src_ref, dst_ref, dl_ref,
    wl1_ref, bl1_ref, wr1_ref, br1_ref, a1_ref, bias1_ref,
    wl2_ref, bl2_ref, wr2_ref, br2_ref, a2_ref, bias2_ref,
    wline_ref, bline_ref, sel_ref, o_ref,
):
    f32 = jnp.float32
    bf16 = jnp.bfloat16

    def mm(a, b):
        return jnp.dot(a, b, preferred_element_type=f32)

    def lrelu(v):
        return jnp.where(v >= 0, v, 0.2 * v)

    p0 = s2_ref[0:1, 0:1]
    p1 = s2_ref[0:1, 1:2]
    t = t_ref[...].reshape(GN, 1)
    pe = jnp.concatenate([p_ref[0]] * GS, axis=0)
    te = (1000.0 * jnp.exp(-p0 * t) + p1) * math.sqrt(float(B)) + pe
    colmask = (lax.broadcasted_iota(jnp.int32, (1, 128), 1) == 127).astype(f32)
    x = x_ref[...].reshape(GN, 128) + te * colmask

    cols = lax.broadcasted_iota(jnp.int32, (EPG, NP), 1)
    rows = lax.broadcasted_iota(jnp.int32, (NP, EPG), 0)

    xl1 = mm(x, wl1_ref[...]) + bl1_ref[...]
    xr1 = mm(x, wr1_ref[...]) + br1_ref[...]
    sl1 = mm(lrelu(xl1 + xr1), a1_ref[...])

    h1_parts = []
    sohs, dohs, masks = [], [], []
    for g in range(GS):
        ns = slice(g * NP, (g + 1) * NP)
        soh = (src_ref[g] == cols).astype(f32)
        doh = (dst_ref[g] == cols).astype(f32)
        mask_t = rows == dl_ref[g]
        mask7 = jnp.concatenate([mask_t] * 7, axis=0)
        sohs.append(soh); dohs.append(doh); masks.append(mask7)
        xl1g = xl1[ns]
        xls = mm(soh, xl1g)
        e = lrelu(xls + mm(doh, xr1[ns]))
        lg_t = mm(e, a1_ref[...]).T
        h1_parts.append(
            _attention(lg_t, sl1[ns], xls, xl1g, mask7, H1, C1, sel_ref[...])
        )
    h1 = jnp.concatenate(h1_parts, axis=0) + bias1_ref[...]
    h1 = jnp.maximum(h1, 0.0)

    xl2 = mm(h1, wl2_ref[...]) + bl2_ref[...]
    xr2 = mm(h1, wr2_ref[...]) + br2_ref[...]
    sl2 = mm(lrelu(xl2 + xr2), a2_ref[...])

    h2_parts = []
    for g in range(GS):
        ns = slice(g * NP, (g + 1) * NP)
        xl2g = xl2[ns]
        xls = mm(sohs[g], xl2g)
        e = lrelu(xls + mm(dohs[g], xr2[ns]))
        lg_t = mm(e, a2_ref[...]).T
        h2_parts.append(
            _attention(lg_t, sl2[ns], xls, xl2g, masks[g], H2, C2, sel_ref[...])
        )
    h2 = jnp.concatenate(h2_parts, axis=0) + bias2_ref[...]

    o = mm(h2, wline_ref[...]) + bline_ref[...]
    o_ref[...] = o.reshape(GS, NP, 64)


def _sel_matrix():
    return ((jnp.arange(7 * NP) // NP)[:, None] == jnp.arange(8)[None, :]).astype(
        jnp.float32
    )


def _att_matrix(att, heads, chead):
    flat = att.reshape(heads * chead)
    head_of = jnp.arange(heads * chead) // chead
    return jnp.where(
        head_of[:, None] == jnp.arange(8)[None, :], flat[:, None], 0.0
    ).astype(jnp.float32)


def _gnn_call(x0, times_sub, p_sub, sc2, src_sub, dst_sub, dst_lane, weights,
              interpret=False):
    (wl1t, bl1, wr1t, br1, a1m, bias1,
     wl2t, bl2, wr2t, br2, a2m, bias2, wlinet, bline, sel) = weights

    def cmap(ndim):
        return lambda b: (0,) * ndim

    in_specs = [
        pl.BlockSpec((GS, NP, 128), lambda b: (b, 0, 0)),
        pl.BlockSpec((GS, NP, 1), lambda b: (b, 0, 0)),
        pl.BlockSpec((1, NP, 1), cmap(3)),
        pl.BlockSpec((1, 8), cmap(2)),
        pl.BlockSpec((GS, EPG, 1), lambda b: (b, 0, 0)),
        pl.BlockSpec((GS, EPG, 1), lambda b: (b, 0, 0)),
        pl.BlockSpec((GS, 1, EPG), lambda b: (b, 0, 0)),
        pl.BlockSpec((128, D1), cmap(2)),
        pl.BlockSpec((1, D1), cmap(2)),
        pl.BlockSpec((128, D1), cmap(2)),
        pl.BlockSpec((1, D1), cmap(2)),
        pl.BlockSpec((D1, 8), cmap(2)),
        pl.BlockSpec((1, D1), cmap(2)),
        pl.BlockSpec((D1, D2), cmap(2)),
        pl.BlockSpec((1, D2), cmap(2)),
        pl.BlockSpec((D1, D2), cmap(2)),
        pl.BlockSpec((1, D2), cmap(2)),
        pl.BlockSpec((D2, 8), cmap(2)),
        pl.BlockSpec((1, D2), cmap(2)),
        pl.BlockSpec((D2, 64), cmap(2)),
        pl.BlockSpec((1, 64), cmap(2)),
        pl.BlockSpec((7 * NP, 8), cmap(2)),
    ]
    out = pl.pallas_call(
        _gnn_body,
        grid=(B // GS,),
        in_specs=in_specs,
        out_specs=pl.BlockSpec((GS, NP, 64), lambda b: (b, 0, 0)),
        out_shape=jax.ShapeDtypeStruct((B, NP, 64), jnp.float32),
        interpret=interpret,
    )(
        x0, times_sub, p_sub, sc2, src_sub, dst_sub, dst_lane,
        wl1t, bl1, wr1t, br1, a1m, bias1,
        wl2t, bl2, wr2t, br2, a2m, bias2, wlinet, bline, sel,
    )
    return out


def kernel(inputs, A_edge, input_times, batch_size, emb_table, p0, p1,
           Wl1, bl1, Wr1, br1, att1, bias1,
           Wl2, bl2, Wr2, br2, att2, bias2, Wline, bline):
    f32 = jnp.float32

    table = jnp.pad(emb_table, ((0, 0), (0, 1)))
    ids = jnp.pad(inputs.astype(jnp.int32), ((0, 0), (0, NP - L)))
    x0 = _sc_gather(table, ids.reshape(1, B * NP)).reshape(B, NP, 128)

    times_sub = jnp.pad(input_times, ((0, 0), (0, NP - L))).reshape(B, NP, 1)
    msl = (1000 // L) * L
    pos = jnp.arange(0, msl, msl // L, dtype=f32)
    p_sub = jnp.pad(jnp.sin(pos), (0, NP - L)).reshape(1, NP, 1)
    sc2 = jnp.concatenate([p0, p1, jnp.zeros((6,), f32)]).reshape(1, 8)
    src_sub = A_edge[:, :, 0].astype(jnp.int32).reshape(B, EPG, 1)
    dst_sub = A_edge[:, :, 1].astype(jnp.int32).reshape(B, EPG, 1)
    dst_lane = A_edge[:, :, 1].astype(jnp.int32).reshape(B, 1, EPG)

    weights = (
        Wl1.T, bl1.reshape(1, D1), Wr1.T, br1.reshape(1, D1),
        _att_matrix(att1, H1, C1), bias1.reshape(1, D1),
        Wl2.T, bl2.reshape(1, D2), Wr2.T, br2.reshape(1, D2),
        _att_matrix(att2, H2, C2), bias2.reshape(1, D2),
        Wline.T, bline.reshape(1, 64),
        _sel_matrix(),
    )
    out = _gnn_call(x0, times_sub, p_sub, sc2, src_sub, dst_sub, dst_lane,
                    weights)
    return out[:, :L, :]


# speedup vs baseline: 1.0362x; 1.0362x over previous
import functools
import math

import jax
import jax.numpy as jnp
from jax import lax
from jax.experimental import pallas as pl
from jax.experimental.pallas import tpu as pltpu
from jax.experimental.pallas import tpu_sc as plsc

B = 512
L = 50
NP = 56
EPG = 128
GS = 8
GN = GS * NP
H1, C1 = 7, 256
H2, C2 = 7, 128
D1 = H1 * C1
D2 = H2 * C2
NEG = -1e30


def _sc_gather(table, ids):
    m = ids.shape[1]
    window = 128
    nwin = m // window
    mesh = plsc.VectorSubcoreMesh(core_axis_name="core", subcore_axis_name="subcore")

    @pl.kernel(
        out_type=jax.ShapeDtypeStruct((m, 128), table.dtype),
        mesh=mesh,
    )
    def gather_kernel(x_hbm, i_hbm, o_hbm):
        def body(i_vmem, o_vmem):
            pltpu.sync_copy(x_hbm.at[i_vmem.at[0]], o_vmem)

        pltpu.emit_pipeline(
            body,
            grid=(2, nwin // 2),
            in_specs=[pl.BlockSpec((1, window),
                                   lambda c, i: (0, c * (nwin // 2) + i))],
            out_specs=[pl.BlockSpec((window, 128),
                                    lambda c, i: (c * (nwin // 2) + i, 0))],
            core_axis_name=("core", "subcore"),
            dimension_semantics=(pltpu.PARALLEL, pltpu.PARALLEL),
        )(i_hbm, o_hbm)

    return gather_kernel(table, ids)


def _attention(logits_t, self_logits, xls, xself, mask7, heads, chead, sel):
    lh7 = jnp.dot(sel, logits_t, preferred_element_type=jnp.float32)
    slt = jnp.concatenate([self_logits] * heads, axis=0)
    ls7 = jnp.sum(slt * sel, axis=1, keepdims=True)
    cand = jnp.where(mask7, lh7, NEG)
    m = jnp.maximum(jnp.max(cand, axis=1, keepdims=True), ls7)
    z = jnp.exp(cand - m)
    zs = jnp.exp(ls7 - m)
    inv = 1.0 / (jnp.sum(z, axis=1, keepdims=True) + zs + 1e-16)
    outs = []
    for h in range(heads):
        rs = slice(h * NP, (h + 1) * NP)
        sl = slice(h * chead, (h + 1) * chead)
        agg = jnp.dot(z[rs], xls[:, sl], preferred_element_type=jnp.float32)
        outs.append(agg * inv[rs] + (zs[rs] * inv[rs]) * xself[:, sl])
    return jnp.concatenate(outs, axis=1)


def _gnn_body(
    x_ref, t_ref, p_ref, s2_ref, src_ref, dst_ref, dl_ref,
    wl1_ref, bl1_ref, wr1_ref, br1_ref, a1_ref, bias1_ref,
    wl2_ref, bl2_ref, wr2_ref, br2_ref, a2_ref, bias2_ref,
    wline_ref, bline_ref, sel_ref, o_ref,
):
    f32 = jnp.float32
    bf16 = jnp.bfloat16

    def mm(a, b):
        return jnp.dot(a, b, preferred_element_type=f32)

    def lrelu(v):
        return jnp.where(v >= 0, v, 0.2 * v)

    p0 = s2_ref[0:1, 0:1]
    p1 = s2_ref[0:1, 1:2]
    t = t_ref[...].reshape(GN, 1)
    pe = jnp.concatenate([p_ref[0]] * GS, axis=0)
    te = (1000.0 * jnp.exp(-p0 * t) + p1) * math.sqrt(float(B)) + pe
    colmask = (lax.broadcasted_iota(jnp.int32, (1, 128), 1) == 127).astype(f32)
    x = x_ref[...].reshape(GN, 128) + te * colmask

    cols = lax.broadcasted_iota(jnp.int32, (EPG, NP), 1)
    rows = lax.broadcasted_iota(jnp.int32, (NP, EPG), 0)

    xl1 = mm(x, wl1_ref[...]) + bl1_ref[...]
    xr1 = mm(x, wr1_ref[...]) + br1_ref[...]
    sl1 = mm(lrelu(xl1 + xr1), a1_ref[...])

    h1_parts = []
    sohs, dohs, masks = [], [], []
    for g in range(GS):
        ns = slice(g * NP, (g + 1) * NP)
        soh = (src_ref[g] == cols).astype(f32)
        doh = (dst_ref[g] == cols).astype(f32)
        mask_t = rows == dl_ref[g]
        mask7 = jnp.concatenate([mask_t] * 7, axis=0)
        sohs.append(soh); dohs.append(doh); masks.append(mask7)
        xl1g = xl1[ns]
        xls = mm(soh, xl1g)
        e = lrelu(xls + mm(doh, xr1[ns]))
        lg_t = mm(e, a1_ref[...]).T
        h1_parts.append(
            _attention(lg_t, sl1[ns], xls, xl1g, mask7, H1, C1, sel_ref[...])
        )
    h1 = jnp.concatenate(h1_parts, axis=0) + bias1_ref[...]
    h1 = jnp.maximum(h1, 0.0)

    xl2 = mm(h1, wl2_ref[...]) + bl2_ref[...]
    xr2 = mm(h1, wr2_ref[...]) + br2_ref[...]
    sl2 = mm(lrelu(xl2 + xr2), a2_ref[...])

    h2_parts = []
    for g in range(GS):
        ns = slice(g * NP, (g + 1) * NP)
        xl2g = xl2[ns]
        xls = mm(sohs[g], xl2g)
        e = lrelu(xls + mm(dohs[g], xr2[ns]))
        lg_t = mm(e, a2_ref[...]).T
        h2_parts.append(
            _attention(lg_t, sl2[ns], xls, xl2g, masks[g], H2, C2, sel_ref[...])
        )
    h2 = jnp.concatenate(h2_parts, axis=0) + bias2_ref[...]

    o = mm(h2, wline_ref[...]) + bline_ref[...]
    o_ref[...] = o.reshape(GS, NP, 64)


def _sel_matrix():
    return ((jnp.arange(7 * NP) // NP)[:, None] == jnp.arange(8)[None, :]).astype(
        jnp.float32
    )


def _att_matrix(att, heads, chead):
    flat = att.reshape(heads * chead)
    head_of = jnp.arange(heads * chead) // chead
    return jnp.where(
        head_of[:, None] == jnp.arange(8)[None, :], flat[:, None], 0.0
    ).astype(jnp.float32)


def _gnn_call(x0, times_sub, p_sub, sc2, src_sub, dst_sub, dst_lane, weights,
              interpret=False):
    (wl1t, bl1, wr1t, br1, a1m, bias1,
     wl2t, bl2, wr2t, br2, a2m, bias2, wlinet, bline, sel) = weights

    def cmap(ndim):
        return lambda b: (0,) * ndim

    in_specs = [
        pl.BlockSpec((GS, NP, 128), lambda b: (b, 0, 0)),
        pl.BlockSpec((GS, NP, 1), lambda b: (b, 0, 0)),
        pl.BlockSpec((1, NP, 1), cmap(3)),
        pl.BlockSpec((1, 8), cmap(2)),
        pl.BlockSpec((GS, EPG, 1), lambda b: (b, 0, 0)),
        pl.BlockSpec((GS, EPG, 1), lambda b: (b, 0, 0)),
        pl.BlockSpec((GS, 1, EPG), lambda b: (b, 0, 0)),
        pl.BlockSpec((128, D1), cmap(2)),
        pl.BlockSpec((1, D1), cmap(2)),
        pl.BlockSpec((128, D1), cmap(2)),
        pl.BlockSpec((1, D1), cmap(2)),
        pl.BlockSpec((D1, 8), cmap(2)),
        pl.BlockSpec((1, D1), cmap(2)),
        pl.BlockSpec((D1, D2), cmap(2)),
        pl.BlockSpec((1, D2), cmap(2)),
        pl.BlockSpec((D1, D2), cmap(2)),
        pl.BlockSpec((1, D2), cmap(2)),
        pl.BlockSpec((D2, 8), cmap(2)),
        pl.BlockSpec((1, D2), cmap(2)),
        pl.BlockSpec((D2, 64), cmap(2)),
        pl.BlockSpec((1, 64), cmap(2)),
        pl.BlockSpec((7 * NP, 8), cmap(2)),
    ]
    nb = x0.shape[0]
    out = pl.pallas_call(
        _gnn_body,
        grid=(nb // GS,),
        in_specs=in_specs,
        out_specs=pl.BlockSpec((GS, NP, 64), lambda b: (b, 0, 0)),
        out_shape=jax.ShapeDtypeStruct((nb, NP, 64), jnp.float32),
        interpret=interpret,
    )(
        x0, times_sub, p_sub, sc2, src_sub, dst_sub, dst_lane,
        wl1t, bl1, wr1t, br1, a1m, bias1,
        wl2t, bl2, wr2t, br2, a2m, bias2, wlinet, bline, sel,
    )
    return out


def kernel(inputs, A_edge, input_times, batch_size, emb_table, p0, p1,
           Wl1, bl1, Wr1, br1, att1, bias1,
           Wl2, bl2, Wr2, br2, att2, bias2, Wline, bline):
    f32 = jnp.float32

    table = jnp.pad(emb_table, ((0, 0), (0, 1)))
    ids = jnp.pad(inputs.astype(jnp.int32), ((0, 0), (0, NP - L)))
    idsf = ids.reshape(1, B * NP)
    half = B * NP // 2
    x0a = _sc_gather(table, idsf[:, :half]).reshape(B // 2, NP, 128)
    x0b = _sc_gather(table, idsf[:, half:]).reshape(B // 2, NP, 128)

    times_sub = jnp.pad(input_times, ((0, 0), (0, NP - L))).reshape(B, NP, 1)
    msl = (1000 // L) * L
    pos = jnp.arange(0, msl, msl // L, dtype=f32)
    p_sub = jnp.pad(jnp.sin(pos), (0, NP - L)).reshape(1, NP, 1)
    sc2 = jnp.concatenate([p0, p1, jnp.zeros((6,), f32)]).reshape(1, 8)
    src_sub = A_edge[:, :, 0].astype(jnp.int32).reshape(B, EPG, 1)
    dst_sub = A_edge[:, :, 1].astype(jnp.int32).reshape(B, EPG, 1)
    dst_lane = A_edge[:, :, 1].astype(jnp.int32).reshape(B, 1, EPG)

    weights = (
        Wl1.T, bl1.reshape(1, D1), Wr1.T, br1.reshape(1, D1),
        _att_matrix(att1, H1, C1), bias1.reshape(1, D1),
        Wl2.T, bl2.reshape(1, D2), Wr2.T, br2.reshape(1, D2),
        _att_matrix(att2, H2, C2), bias2.reshape(1, D2),
        Wline.T, bline.reshape(1, 64),
        _sel_matrix(),
    )
    hb = B // 2
    outa = _gnn_call(x0a, times_sub[:hb], p_sub, sc2, src_sub[:hb],
                     dst_sub[:hb], dst_lane[:hb], weights)
    outb = _gnn_call(x0b, times_sub[hb:], p_sub, sc2, src_sub[hb:],
                     dst_sub[hb:], dst_lane[hb:], weights)
    return jnp.concatenate([outa, outb], axis=0)[:, :L, :]
